# Initial kernel scaffold; baseline (speedup 1.0000x reference)
import jax
import jax.numpy as jnp
from jax.experimental import pallas as pl


def kernel(x, edge_index, W1, b1, W2, b2, Wr1, br1, Wr2, br2, Ws1, bs1, Ws2, bs2):
    raise NotImplementedError("write your pallas kernel here")


# speedup vs baseline: 27.9487x; 27.9487x over previous
import functools

import jax
import jax.numpy as jnp
from jax import lax
from jax.experimental import pallas as pl
from jax.experimental.pallas import tpu as pltpu, tpu_sc as plsc

N = 10000
E = 320000
D_IN = 128
D_H = 64
D_OUT = 32

NC = 2
NS = 16
NW = NC * NS
EPW = E // NW
CHUNK = 128
NCH = EPW // CHUNK
TAIL = EPW - NCH * CHUNK
ROWS_PER_TILE = 624
REM_ROWS = N - NS * ROWS_PER_TILE


def _sc_mesh():
    return plsc.VectorSubcoreMesh(core_axis_name="c", subcore_axis_name="s")


def _make_deg_kernel():
    @functools.partial(
        pl.kernel,
        out_type=jax.ShapeDtypeStruct((NC, N, 8), jnp.float32),
        mesh=_sc_mesh(),
        scratch_types=[
            pltpu.VMEM((NCH, CHUNK), jnp.int32),
            pltpu.VMEM((1, TAIL), jnp.int32),
            pltpu.VMEM((CHUNK, 8), jnp.float32),
            pltpu.VMEM_SHARED((N, 8), jnp.float32),
        ],
        compiler_params=pltpu.CompilerParams(use_tc_tiling_on_sc=False),
    )
    def deg_kernel(dstm, dstt, ones, zeros, out, dstv, dsttv, onesv, acc):
        c = lax.axis_index("c")
        s = lax.axis_index("s")
        w = s * NC + c

        @pl.when(s == 0)
        def _():
            pltpu.sync_copy(zeros, acc)

        pltpu.sync_copy(dstm.at[w], dstv)
        pltpu.sync_copy(dstt.at[w], dsttv)
        pltpu.sync_copy(ones, onesv)
        plsc.subcore_barrier()

        def body(j, carry):
            pltpu.sync_copy(onesv, acc.at[dstv.at[j]], add=True)
            return carry

        lax.fori_loop(0, NCH, body, 0)
        pltpu.sync_copy(onesv.at[pl.ds(0, TAIL)], acc.at[dsttv.at[0]], add=True)
        plsc.subcore_barrier()
        _writeback(acc, out, c, s)

    return deg_kernel


def _writeback(acc, out, c, s):
    pltpu.sync_copy(
        acc.at[pl.ds(s * ROWS_PER_TILE, ROWS_PER_TILE)],
        out.at[c, pl.ds(s * ROWS_PER_TILE, ROWS_PER_TILE)],
    )

    @pl.when(s == NS - 1)
    def _():
        pltpu.sync_copy(
            acc.at[pl.ds(NS * ROWS_PER_TILE, REM_ROWS)],
            out.at[c, pl.ds(NS * ROWS_PER_TILE, REM_ROWS)],
        )


def _make_agg_kernel(d):
    @functools.partial(
        pl.kernel,
        out_type=jax.ShapeDtypeStruct((NC, N, d), jnp.float32),
        mesh=_sc_mesh(),
        scratch_types=[
            pltpu.VMEM((NCH, CHUNK), jnp.int32),
            pltpu.VMEM((NCH, CHUNK), jnp.int32),
            pltpu.VMEM((1, TAIL), jnp.int32),
            pltpu.VMEM((1, TAIL), jnp.int32),
            pltpu.VMEM((CHUNK, d), jnp.float32),
            pltpu.VMEM((TAIL, d), jnp.float32),
            pltpu.VMEM_SHARED((N, d), jnp.float32),
            pltpu.SemaphoreType.DMA,
        ],
        compiler_params=pltpu.CompilerParams(use_tc_tiling_on_sc=False),
    )
    def agg_kernel(y, srcm, dstm, srct, dstt, zeros, out,
                   srcv, dstv, srctv, dsttv, buf, buft, acc, sem):
        c = lax.axis_index("c")
        s = lax.axis_index("s")
        w = s * NC + c

        @pl.when(s == 0)
        def _():
            pltpu.sync_copy(zeros, acc)

        pltpu.sync_copy(srcm.at[w], srcv)
        pltpu.sync_copy(dstm.at[w], dstv)
        pltpu.sync_copy(srct.at[w], srctv)
        pltpu.sync_copy(dstt.at[w], dsttv)
        plsc.subcore_barrier()

        def body(j, carry):
            pltpu.async_copy(y.at[srcv.at[j]], buf, sem).wait()
            pltpu.sync_copy(buf, acc.at[dstv.at[j]], add=True)
            return carry

        lax.fori_loop(0, NCH, body, 0)
        pltpu.async_copy(y.at[srctv.at[0]], buft, sem).wait()
        pltpu.sync_copy(buft, acc.at[dsttv.at[0]], add=True)
        plsc.subcore_barrier()
        _writeback(acc, out, c, s)

    return agg_kernel


_deg_kernel = _make_deg_kernel()
_agg64 = _make_agg_kernel(D_H)
_agg32 = _make_agg_kernel(D_OUT)

_R = 2000


def _dot(a, b):
    return jnp.dot(a, b, preferred_element_type=jnp.float32,
                   precision=lax.Precision.HIGHEST)


def _tc1_body(degp, x, w1, dis_o, y1_o):
    deg = degp[0, :, 0:1] + degp[1, :, 0:1] + 1.0
    dis = lax.rsqrt(deg)
    dis_o[...] = dis
    y1_o[...] = dis * _dot(x[...], w1[...])


def _tc1(degp, x, w1):
    return pl.pallas_call(
        _tc1_body,
        grid=(N // _R,),
        in_specs=[
            pl.BlockSpec((NC, _R, 8), lambda i: (0, i, 0)),
            pl.BlockSpec((_R, D_IN), lambda i: (i, 0)),
            pl.BlockSpec((D_IN, D_H), lambda i: (0, 0)),
        ],
        out_specs=[
            pl.BlockSpec((_R, 1), lambda i: (i, 0)),
            pl.BlockSpec((_R, D_H), lambda i: (i, 0)),
        ],
        out_shape=[
            jax.ShapeDtypeStruct((N, 1), jnp.float32),
            jax.ShapeDtypeStruct((N, D_H), jnp.float32),
        ],
    )(degp, x, w1)


def _tc2_body(agg, y1, dis, b1, w2, y2_o):
    h1 = jax.nn.relu(dis[...] * (agg[0] + agg[1] + y1[...]) + b1[...])
    y2_o[...] = dis[...] * _dot(h1, w2[...])


def _tc2(agg, y1, dis, b1, w2):
    return pl.pallas_call(
        _tc2_body,
        grid=(N // _R,),
        in_specs=[
            pl.BlockSpec((NC, _R, D_H), lambda i: (0, i, 0)),
            pl.BlockSpec((_R, D_H), lambda i: (i, 0)),
            pl.BlockSpec((_R, 1), lambda i: (i, 0)),
            pl.BlockSpec((1, D_H), lambda i: (0, 0)),
            pl.BlockSpec((D_H, D_OUT), lambda i: (0, 0)),
        ],
        out_specs=pl.BlockSpec((_R, D_OUT), lambda i: (i, 0)),
        out_shape=jax.ShapeDtypeStruct((N, D_OUT), jnp.float32),
    )(agg, y1, dis, b1, w2)


def _tc3_body(agg, y2, dis, b2, wr1, br1, wr2, br2, ws1, bs1, ws2, bs2,
              h_o, recon_o, score_o):
    h = dis[...] * (agg[0] + agg[1] + y2[...]) + b2[...]
    h_o[...] = h
    r = jax.nn.relu(_dot(h, wr1[...]) + br1[...])
    recon_o[...] = _dot(r, wr2[...]) + br2[...]
    sc = jax.nn.relu(_dot(h, ws1[...]) + bs1[...])
    score_o[...] = jax.nn.sigmoid(_dot(sc, ws2[...]) + bs2[...])


def _tc3(agg, y2, dis, b2, wr1, br1, wr2, br2, ws1, bs1, ws2, bs2):
    row = lambda i: (i, 0)
    full = lambda i: (0, 0)
    return pl.pallas_call(
        _tc3_body,
        grid=(N // _R,),
        in_specs=[
            pl.BlockSpec((NC, _R, D_OUT), lambda i: (0, i, 0)),
            pl.BlockSpec((_R, D_OUT), row),
            pl.BlockSpec((_R, 1), row),
            pl.BlockSpec((1, D_OUT), full),
            pl.BlockSpec((D_OUT, D_H), full),
            pl.BlockSpec((1, D_H), full),
            pl.BlockSpec((D_H, D_IN), full),
            pl.BlockSpec((1, D_IN), full),
            pl.BlockSpec((D_OUT, D_H // 2), full),
            pl.BlockSpec((1, D_H // 2), full),
            pl.BlockSpec((D_H // 2, 1), full),
            pl.BlockSpec((1, 1), full),
        ],
        out_specs=[
            pl.BlockSpec((_R, D_OUT), row),
            pl.BlockSpec((_R, D_IN), row),
            pl.BlockSpec((_R, 1), row),
        ],
        out_shape=[
            jax.ShapeDtypeStruct((N, D_OUT), jnp.float32),
            jax.ShapeDtypeStruct((N, D_IN), jnp.float32),
            jax.ShapeDtypeStruct((N, 1), jnp.float32),
        ],
    )(agg, y2, dis, b2, wr1, br1, wr2, br2, ws1, bs1, ws2, bs2)


def _split_edges(idx):
    per_w = idx.reshape(NW, EPW)
    main = per_w[:, : NCH * CHUNK].reshape(NW, NCH, CHUNK)
    tail = per_w[:, NCH * CHUNK:].reshape(NW, 1, TAIL)
    return main, tail


def kernel(x, edge_index, W1, b1, W2, b2, Wr1, br1, Wr2, br2, Ws1, bs1, Ws2, bs2):
    srcm, srct = _split_edges(edge_index[0])
    dstm, dstt = _split_edges(edge_index[1])

    degp = _deg_kernel(dstm, dstt,
                       jnp.ones((CHUNK, 8), jnp.float32),
                       jnp.zeros((N, 8), jnp.float32))
    dis, y1 = _tc1(degp, x, W1)

    agg1 = _agg64(y1, srcm, dstm, srct, dstt, jnp.zeros((N, D_H), jnp.float32))
    y2 = _tc2(agg1, y1, dis, b1.reshape(1, D_H), W2)

    agg2 = _agg32(y2, srcm, dstm, srct, dstt, jnp.zeros((N, D_OUT), jnp.float32))
    h, recon, score = _tc3(agg2, y2, dis, b2.reshape(1, D_OUT),
                           Wr1, br1.reshape(1, D_H), Wr2, br2.reshape(1, D_IN),
                           Ws1, bs1.reshape(1, D_H // 2), Ws2, bs2.reshape(1, 1))
    return (h, recon, score)


# speedup vs baseline: 41.2399x; 1.4756x over previous
import functools

import jax
import jax.numpy as jnp
from jax import lax
from jax.experimental import pallas as pl
from jax.experimental.pallas import tpu as pltpu, tpu_sc as plsc

N = 10000
E = 320000
D_IN = 128
D_H = 64
D_OUT = 32

NC = 2
NS = 16
NW = NC * NS
EPW = E // NW
CHUNK = 128
NCH = EPW // CHUNK
TAIL = EPW - NCH * CHUNK
NBUF = 3
ROWS_PER_TILE = 624
REM_ROWS = N - NS * ROWS_PER_TILE


def _sc_mesh():
    return plsc.VectorSubcoreMesh(core_axis_name="c", subcore_axis_name="s")


def _make_deg_kernel():
    @functools.partial(
        pl.kernel,
        out_type=jax.ShapeDtypeStruct((NC, N, 8), jnp.float32),
        mesh=_sc_mesh(),
        scratch_types=[
            pltpu.VMEM((NCH, CHUNK), jnp.int32),
            pltpu.VMEM((1, TAIL), jnp.int32),
            pltpu.VMEM((CHUNK, 8), jnp.float32),
            pltpu.VMEM_SHARED((N, 8), jnp.float32),
            pltpu.SemaphoreType.DMA,
        ],
        compiler_params=pltpu.CompilerParams(use_tc_tiling_on_sc=False),
    )
    def deg_kernel(dstm, dstt, ones, zeros, out, dstv, dsttv, onesv, acc, sem):
        c = lax.axis_index("c")
        s = lax.axis_index("s")
        w = s * NC + c

        @pl.when(s == 0)
        def _():
            pltpu.sync_copy(zeros, acc)

        pltpu.sync_copy(dstm.at[w], dstv)
        pltpu.sync_copy(dstt.at[w], dsttv)
        pltpu.sync_copy(ones, onesv)
        plsc.subcore_barrier()

        def body(j, carry):
            pltpu.async_copy(onesv, acc.at[dstv.at[j]], sem, add=True)
            return carry

        lax.fori_loop(0, NCH, body, 0)

        def drain(j, carry):
            pltpu.make_async_copy(onesv, acc.at[dstv.at[j]], sem).wait()
            return carry

        lax.fori_loop(0, NCH, drain, 0)
        pltpu.sync_copy(onesv.at[pl.ds(0, TAIL)], acc.at[dsttv.at[0]], add=True)
        plsc.subcore_barrier()
        _writeback(acc, out, c, s)

    return deg_kernel


def _writeback(acc, out, c, s):
    pltpu.sync_copy(
        acc.at[pl.ds(s * ROWS_PER_TILE, ROWS_PER_TILE)],
        out.at[c, pl.ds(s * ROWS_PER_TILE, ROWS_PER_TILE)],
    )

    @pl.when(s == NS - 1)
    def _():
        pltpu.sync_copy(
            acc.at[pl.ds(NS * ROWS_PER_TILE, REM_ROWS)],
            out.at[c, pl.ds(NS * ROWS_PER_TILE, REM_ROWS)],
        )


def _make_agg_kernel(d):
    @functools.partial(
        pl.kernel,
        out_type=jax.ShapeDtypeStruct((NC, N, d), jnp.float32),
        mesh=_sc_mesh(),
        scratch_types=[
            pltpu.VMEM((NCH, CHUNK), jnp.int32),
            pltpu.VMEM((NCH, CHUNK), jnp.int32),
            pltpu.VMEM((1, TAIL), jnp.int32),
            pltpu.VMEM((1, TAIL), jnp.int32),
            pltpu.VMEM((NBUF, CHUNK, d), jnp.float32),
            pltpu.VMEM((TAIL, d), jnp.float32),
            pltpu.VMEM_SHARED((N, d), jnp.float32),
            pltpu.SemaphoreType.DMA((NBUF,)),
        ],
        compiler_params=pltpu.CompilerParams(use_tc_tiling_on_sc=False),
    )
    def agg_kernel(y, srcm, dstm, srct, dstt, zeros, out,
                   srcv, dstv, srctv, dsttv, buf, buft, acc, sem):
        c = lax.axis_index("c")
        s = lax.axis_index("s")
        w = s * NC + c

        @pl.when(s == 0)
        def _():
            pltpu.sync_copy(zeros, acc)

        pltpu.sync_copy(srcm.at[w], srcv)
        pltpu.sync_copy(dstm.at[w], dstv)
        pltpu.sync_copy(srct.at[w], srctv)
        pltpu.sync_copy(dstt.at[w], dsttv)
        plsc.subcore_barrier()

        for b in range(NBUF):
            pltpu.async_copy(y.at[srcv.at[b]], buf.at[b], sem.at[b])

        def body(g, carry):
            for b in range(NBUF):
                j = g * NBUF + b
                pltpu.make_async_copy(y.at[srcv.at[j]], buf.at[b], sem.at[b]).wait()
                pltpu.sync_copy(buf.at[b], acc.at[dstv.at[j]], add=True)

                @pl.when(j + NBUF < NCH)
                def _():
                    pltpu.async_copy(y.at[srcv.at[j + NBUF]], buf.at[b], sem.at[b])

            return carry

        lax.fori_loop(0, NCH // NBUF, body, 0)
        pltpu.async_copy(y.at[srctv.at[0]], buft, sem.at[0]).wait()
        pltpu.sync_copy(buft, acc.at[dsttv.at[0]], add=True)
        plsc.subcore_barrier()
        _writeback(acc, out, c, s)

    return agg_kernel


_deg_kernel = _make_deg_kernel()
_agg64 = _make_agg_kernel(D_H)
_agg32 = _make_agg_kernel(D_OUT)

_R = 2000


def _dot(a, b):
    return jnp.dot(a, b, preferred_element_type=jnp.float32,
                   precision=lax.Precision.HIGHEST)


def _tc1_body(degp, x, w1, dis_o, y1_o):
    deg = degp[0, :, 0:1] + degp[1, :, 0:1] + 1.0
    dis = lax.rsqrt(deg)
    dis_o[...] = dis
    y1_o[...] = dis * _dot(x[...], w1[...])


def _tc1(degp, x, w1):
    return pl.pallas_call(
        _tc1_body,
        grid=(N // _R,),
        in_specs=[
            pl.BlockSpec((NC, _R, 8), lambda i: (0, i, 0)),
            pl.BlockSpec((_R, D_IN), lambda i: (i, 0)),
            pl.BlockSpec((D_IN, D_H), lambda i: (0, 0)),
        ],
        out_specs=[
            pl.BlockSpec((_R, 1), lambda i: (i, 0)),
            pl.BlockSpec((_R, D_H), lambda i: (i, 0)),
        ],
        out_shape=[
            jax.ShapeDtypeStruct((N, 1), jnp.float32),
            jax.ShapeDtypeStruct((N, D_H), jnp.float32),
        ],
    )(degp, x, w1)


def _tc2_body(agg, y1, dis, b1, w2, y2_o):
    h1 = jax.nn.relu(dis[...] * (agg[0] + agg[1] + y1[...]) + b1[...])
    y2_o[...] = dis[...] * _dot(h1, w2[...])


def _tc2(agg, y1, dis, b1, w2):
    return pl.pallas_call(
        _tc2_body,
        grid=(N // _R,),
        in_specs=[
            pl.BlockSpec((NC, _R, D_H), lambda i: (0, i, 0)),
            pl.BlockSpec((_R, D_H), lambda i: (i, 0)),
            pl.BlockSpec((_R, 1), lambda i: (i, 0)),
            pl.BlockSpec((1, D_H), lambda i: (0, 0)),
            pl.BlockSpec((D_H, D_OUT), lambda i: (0, 0)),
        ],
        out_specs=pl.BlockSpec((_R, D_OUT), lambda i: (i, 0)),
        out_shape=jax.ShapeDtypeStruct((N, D_OUT), jnp.float32),
    )(agg, y1, dis, b1, w2)


def _tc3_body(agg, y2, dis, b2, wr1, br1, wr2, br2, ws1, bs1, ws2, bs2,
              h_o, recon_o, score_o):
    h = dis[...] * (agg[0] + agg[1] + y2[...]) + b2[...]
    h_o[...] = h
    r = jax.nn.relu(_dot(h, wr1[...]) + br1[...])
    recon_o[...] = _dot(r, wr2[...]) + br2[...]
    sc = jax.nn.relu(_dot(h, ws1[...]) + bs1[...])
    score_o[...] = jax.nn.sigmoid(_dot(sc, ws2[...]) + bs2[...])


def _tc3(agg, y2, dis, b2, wr1, br1, wr2, br2, ws1, bs1, ws2, bs2):
    row = lambda i: (i, 0)
    full = lambda i: (0, 0)
    return pl.pallas_call(
        _tc3_body,
        grid=(N // _R,),
        in_specs=[
            pl.BlockSpec((NC, _R, D_OUT), lambda i: (0, i, 0)),
            pl.BlockSpec((_R, D_OUT), row),
            pl.BlockSpec((_R, 1), row),
            pl.BlockSpec((1, D_OUT), full),
            pl.BlockSpec((D_OUT, D_H), full),
            pl.BlockSpec((1, D_H), full),
            pl.BlockSpec((D_H, D_IN), full),
            pl.BlockSpec((1, D_IN), full),
            pl.BlockSpec((D_OUT, D_H // 2), full),
            pl.BlockSpec((1, D_H // 2), full),
            pl.BlockSpec((D_H // 2, 1), full),
            pl.BlockSpec((1, 1), full),
        ],
        out_specs=[
            pl.BlockSpec((_R, D_OUT), row),
            pl.BlockSpec((_R, D_IN), row),
            pl.BlockSpec((_R, 1), row),
        ],
        out_shape=[
            jax.ShapeDtypeStruct((N, D_OUT), jnp.float32),
            jax.ShapeDtypeStruct((N, D_IN), jnp.float32),
            jax.ShapeDtypeStruct((N, 1), jnp.float32),
        ],
    )(agg, y2, dis, b2, wr1, br1, wr2, br2, ws1, bs1, ws2, bs2)


def _split_edges(idx):
    per_w = idx.reshape(NW, EPW)
    main = per_w[:, : NCH * CHUNK].reshape(NW, NCH, CHUNK)
    tail = per_w[:, NCH * CHUNK:].reshape(NW, 1, TAIL)
    return main, tail


def kernel(x, edge_index, W1, b1, W2, b2, Wr1, br1, Wr2, br2, Ws1, bs1, Ws2, bs2):
    srcm, srct = _split_edges(edge_index[0])
    dstm, dstt = _split_edges(edge_index[1])

    degp = _deg_kernel(dstm, dstt,
                       jnp.ones((CHUNK, 8), jnp.float32),
                       jnp.zeros((N, 8), jnp.float32))
    dis, y1 = _tc1(degp, x, W1)

    agg1 = _agg64(y1, srcm, dstm, srct, dstt, jnp.zeros((N, D_H), jnp.float32))
    y2 = _tc2(agg1, y1, dis, b1.reshape(1, D_H), W2)

    agg2 = _agg32(y2, srcm, dstm, srct, dstt, jnp.zeros((N, D_OUT), jnp.float32))
    h, recon, score = _tc3(agg2, y2, dis, b2.reshape(1, D_OUT),
                           Wr1, br1.reshape(1, D_H), Wr2, br2.reshape(1, D_IN),
                           Ws1, bs1.reshape(1, D_H // 2), Ws2, bs2.reshape(1, 1))
    return (h, recon, score)


# speedup vs baseline: 49.1082x; 1.1908x over previous
import functools

import jax
import jax.numpy as jnp
from jax import lax
from jax.experimental import pallas as pl
from jax.experimental.pallas import tpu as pltpu, tpu_sc as plsc

N = 10000
E = 320000
D_IN = 128
D_H = 64
D_OUT = 32

NC = 2
NS = 16
NW = NC * NS
CHUNK = 128
NCHUNKS = E // CHUNK
BASE = NCHUNKS // NW
EXTRA = NCHUNKS - BASE * NW
NBUF = 4
ROWS_PER_TILE = 624
REM_ROWS = N - NS * ROWS_PER_TILE


def _sc_mesh():
    return plsc.VectorSubcoreMesh(core_axis_name="c", subcore_axis_name="s")


def _worker_slab(c, s):
    w = s * NC + c
    start = BASE * w + jnp.minimum(w, EXTRA)
    has_extra = w < EXTRA
    return w, start, has_extra


def _load_idx_slab(ei, row, start, has_extra, idxv):
    pltpu.sync_copy(ei.at[row, pl.ds(start, BASE)], idxv.at[pl.ds(0, BASE)])

    @pl.when(has_extra)
    def _():
        pltpu.sync_copy(ei.at[row, pl.ds(start + BASE, 1)],
                        idxv.at[pl.ds(BASE, 1)])


def _writeback(acc, out, c, s):
    pltpu.sync_copy(
        acc.at[pl.ds(s * ROWS_PER_TILE, ROWS_PER_TILE)],
        out.at[c, pl.ds(s * ROWS_PER_TILE, ROWS_PER_TILE)],
    )

    @pl.when(s == NS - 1)
    def _():
        pltpu.sync_copy(
            acc.at[pl.ds(NS * ROWS_PER_TILE, REM_ROWS)],
            out.at[c, pl.ds(NS * ROWS_PER_TILE, REM_ROWS)],
        )


def _make_deg_kernel():
    @functools.partial(
        pl.kernel,
        out_type=jax.ShapeDtypeStruct((NC, N, 8), jnp.float32),
        mesh=_sc_mesh(),
        scratch_types=[
            pltpu.VMEM((BASE + 1, CHUNK), jnp.int32),
            pltpu.VMEM((CHUNK, 8), jnp.float32),
            pltpu.VMEM_SHARED((N, 8), jnp.float32),
            pltpu.SemaphoreType.DMA,
        ],
        compiler_params=pltpu.CompilerParams(use_tc_tiling_on_sc=False),
    )
    def deg_kernel(ei, ones, zeros, out, dstv, onesv, acc, sem):
        c = lax.axis_index("c")
        s = lax.axis_index("s")
        w, start, has_extra = _worker_slab(c, s)

        @pl.when(s == 0)
        def _():
            pltpu.sync_copy(zeros, acc)

        _load_idx_slab(ei, 1, start, has_extra, dstv)
        pltpu.sync_copy(ones, onesv)
        plsc.subcore_barrier()

        def fire(j, carry):
            pltpu.async_copy(onesv, acc.at[dstv.at[j]], sem, add=True)
            return carry

        lax.fori_loop(0, BASE, fire, 0)

        @pl.when(has_extra)
        def _():
            pltpu.async_copy(onesv, acc.at[dstv.at[BASE]], sem, add=True)

        def drain(j, carry):
            pltpu.make_async_copy(onesv, acc.at[dstv.at[j]], sem).wait()
            return carry

        lax.fori_loop(0, BASE, drain, 0)

        @pl.when(has_extra)
        def _():
            pltpu.make_async_copy(onesv, acc.at[dstv.at[BASE]], sem).wait()

        plsc.subcore_barrier()
        _writeback(acc, out, c, s)

    return deg_kernel


def _make_agg_kernel(d):
    @functools.partial(
        pl.kernel,
        out_type=jax.ShapeDtypeStruct((NC, N, d), jnp.float32),
        mesh=_sc_mesh(),
        scratch_types=[
            pltpu.VMEM((BASE + 1, CHUNK), jnp.int32),
            pltpu.VMEM((BASE + 1, CHUNK), jnp.int32),
            pltpu.VMEM((NBUF, CHUNK, d), jnp.float32),
            pltpu.VMEM_SHARED((N, d), jnp.float32),
            pltpu.SemaphoreType.DMA((NBUF,)),
        ],
        compiler_params=pltpu.CompilerParams(use_tc_tiling_on_sc=False),
    )
    def agg_kernel(y, ei, zeros, out, srcv, dstv, buf, acc, sem):
        c = lax.axis_index("c")
        s = lax.axis_index("s")
        w, start, has_extra = _worker_slab(c, s)
        nch = BASE + has_extra.astype(jnp.int32)

        @pl.when(s == 0)
        def _():
            pltpu.sync_copy(zeros, acc)

        _load_idx_slab(ei, 0, start, has_extra, srcv)
        _load_idx_slab(ei, 1, start, has_extra, dstv)
        plsc.subcore_barrier()

        for b in range(NBUF):
            pltpu.async_copy(y.at[srcv.at[b]], buf.at[b], sem.at[b])

        def body(g, carry):
            for b in range(NBUF):
                j = g * NBUF + b

                @pl.when(j < nch)
                def _(b=b, j=j):
                    pltpu.make_async_copy(
                        y.at[srcv.at[j]], buf.at[b], sem.at[b]).wait()
                    pltpu.sync_copy(buf.at[b], acc.at[dstv.at[j]], add=True)

                @pl.when(j + NBUF < nch)
                def _(b=b, j=j):
                    pltpu.async_copy(
                        y.at[srcv.at[j + NBUF]], buf.at[b], sem.at[b])

            return carry

        lax.fori_loop(0, (BASE + NBUF) // NBUF, body, 0)
        plsc.subcore_barrier()
        _writeback(acc, out, c, s)

    return agg_kernel


_deg_kernel = _make_deg_kernel()
_agg64 = _make_agg_kernel(D_H)
_agg32 = _make_agg_kernel(D_OUT)

_R = 2000


def _dot(a, b):
    return jnp.dot(a, b, preferred_element_type=jnp.float32)


def _tc1_body(degp, x, w1, dis_o, y1_o):
    deg = degp[0, :, 0:1] + degp[1, :, 0:1] + 1.0
    dis = lax.rsqrt(deg)
    dis_o[...] = dis
    y1_o[...] = dis * _dot(x[...], w1[...])


def _tc1(degp, x, w1):
    return pl.pallas_call(
        _tc1_body,
        grid=(N // _R,),
        in_specs=[
            pl.BlockSpec((NC, _R, 8), lambda i: (0, i, 0)),
            pl.BlockSpec((_R, D_IN), lambda i: (i, 0)),
            pl.BlockSpec((D_IN, D_H), lambda i: (0, 0)),
        ],
        out_specs=[
            pl.BlockSpec((_R, 1), lambda i: (i, 0)),
            pl.BlockSpec((_R, D_H), lambda i: (i, 0)),
        ],
        out_shape=[
            jax.ShapeDtypeStruct((N, 1), jnp.float32),
            jax.ShapeDtypeStruct((N, D_H), jnp.float32),
        ],
    )(degp, x, w1)


def _tc2_body(agg, y1, dis, b1, w2, y2_o):
    h1 = jax.nn.relu(dis[...] * (agg[0] + agg[1] + y1[...]) + b1[...])
    y2_o[...] = dis[...] * _dot(h1, w2[...])


def _tc2(agg, y1, dis, b1, w2):
    return pl.pallas_call(
        _tc2_body,
        grid=(N // _R,),
        in_specs=[
            pl.BlockSpec((NC, _R, D_H), lambda i: (0, i, 0)),
            pl.BlockSpec((_R, D_H), lambda i: (i, 0)),
            pl.BlockSpec((_R, 1), lambda i: (i, 0)),
            pl.BlockSpec((1, D_H), lambda i: (0, 0)),
            pl.BlockSpec((D_H, D_OUT), lambda i: (0, 0)),
        ],
        out_specs=pl.BlockSpec((_R, D_OUT), lambda i: (i, 0)),
        out_shape=jax.ShapeDtypeStruct((N, D_OUT), jnp.float32),
    )(agg, y1, dis, b1, w2)


def _tc3_body(agg, y2, dis, b2, wr1, br1, wr2, br2, ws1, bs1, ws2, bs2,
              h_o, recon_o, score_o):
    h = dis[...] * (agg[0] + agg[1] + y2[...]) + b2[...]
    h_o[...] = h
    r = jax.nn.relu(_dot(h, wr1[...]) + br1[...])
    recon_o[...] = _dot(r, wr2[...]) + br2[...]
    sc = jax.nn.relu(_dot(h, ws1[...]) + bs1[...])
    score_o[...] = jax.nn.sigmoid(_dot(sc, ws2[...]) + bs2[...])


def _tc3(agg, y2, dis, b2, wr1, br1, wr2, br2, ws1, bs1, ws2, bs2):
    row = lambda i: (i, 0)
    full = lambda i: (0, 0)
    return pl.pallas_call(
        _tc3_body,
        grid=(N // _R,),
        in_specs=[
            pl.BlockSpec((NC, _R, D_OUT), lambda i: (0, i, 0)),
            pl.BlockSpec((_R, D_OUT), row),
            pl.BlockSpec((_R, 1), row),
            pl.BlockSpec((1, D_OUT), full),
            pl.BlockSpec((D_OUT, D_H), full),
            pl.BlockSpec((1, D_H), full),
            pl.BlockSpec((D_H, D_IN), full),
            pl.BlockSpec((1, D_IN), full),
            pl.BlockSpec((D_OUT, D_H // 2), full),
            pl.BlockSpec((1, D_H // 2), full),
            pl.BlockSpec((D_H // 2, 1), full),
            pl.BlockSpec((1, 1), full),
        ],
        out_specs=[
            pl.BlockSpec((_R, D_OUT), row),
            pl.BlockSpec((_R, D_IN), row),
            pl.BlockSpec((_R, 1), row),
        ],
        out_shape=[
            jax.ShapeDtypeStruct((N, D_OUT), jnp.float32),
            jax.ShapeDtypeStruct((N, D_IN), jnp.float32),
            jax.ShapeDtypeStruct((N, 1), jnp.float32),
        ],
    )(agg, y2, dis, b2, wr1, br1, wr2, br2, ws1, bs1, ws2, bs2)


def kernel(x, edge_index, W1, b1, W2, b2, Wr1, br1, Wr2, br2, Ws1, bs1, Ws2, bs2):
    ei = edge_index.reshape(2, NCHUNKS, CHUNK)

    degp = _deg_kernel(ei,
                       jnp.ones((CHUNK, 8), jnp.float32),
                       jnp.zeros((N, 8), jnp.float32))
    dis, y1 = _tc1(degp, x, W1)

    agg1 = _agg64(y1, ei, jnp.zeros((N, D_H), jnp.float32))
    y2 = _tc2(agg1, y1, dis, b1.reshape(1, D_H), W2)

    agg2 = _agg32(y2, ei, jnp.zeros((N, D_OUT), jnp.float32))
    h, recon, score = _tc3(agg2, y2, dis, b2.reshape(1, D_OUT),
                           Wr1, br1.reshape(1, D_H), Wr2, br2.reshape(1, D_IN),
                           Ws1, bs1.reshape(1, D_H // 2), Ws2, bs2.reshape(1, 1))
    return (h, recon, score)


# speedup vs baseline: 49.2074x; 1.0020x over previous
import functools

import jax
import jax.numpy as jnp
from jax import lax
from jax.experimental import pallas as pl
from jax.experimental.pallas import tpu as pltpu, tpu_sc as plsc

N = 10000
E = 320000
D_IN = 128
D_H = 64
D_OUT = 32

NC = 2
NS = 16
NW = NC * NS
CHUNK = 128
NCHUNKS = E // CHUNK
BASE = NCHUNKS // NW
EXTRA = NCHUNKS - BASE * NW
NBUF = 4
ROWS_PER_TILE = 624
REM_ROWS = N - NS * ROWS_PER_TILE


def _sc_mesh():
    return plsc.VectorSubcoreMesh(core_axis_name="c", subcore_axis_name="s")


def _worker_slab(c, s):
    w = s * NC + c
    start = BASE * w + jnp.minimum(w, EXTRA)
    has_extra = w < EXTRA
    return w, start, has_extra


def _load_idx_slab(ei, row, start, has_extra, idxv):
    pltpu.sync_copy(ei.at[row, pl.ds(start, BASE)], idxv.at[pl.ds(0, BASE)])

    @pl.when(has_extra)
    def _():
        pltpu.sync_copy(ei.at[row, pl.ds(start + BASE, 1)],
                        idxv.at[pl.ds(BASE, 1)])


def _writeback(acc, out, c, s):
    pltpu.sync_copy(
        acc.at[pl.ds(s * ROWS_PER_TILE, ROWS_PER_TILE)],
        out.at[c, pl.ds(s * ROWS_PER_TILE, ROWS_PER_TILE)],
    )

    @pl.when(s == NS - 1)
    def _():
        pltpu.sync_copy(
            acc.at[pl.ds(NS * ROWS_PER_TILE, REM_ROWS)],
            out.at[c, pl.ds(NS * ROWS_PER_TILE, REM_ROWS)],
        )


def _make_deg_kernel():
    @functools.partial(
        pl.kernel,
        out_type=jax.ShapeDtypeStruct((NC, N, 8), jnp.float32),
        mesh=_sc_mesh(),
        scratch_types=[
            pltpu.VMEM((BASE + 1, CHUNK), jnp.int32),
            pltpu.VMEM((CHUNK, 8), jnp.float32),
            pltpu.VMEM_SHARED((N, 8), jnp.float32),
            pltpu.SemaphoreType.DMA,
        ],
        compiler_params=pltpu.CompilerParams(use_tc_tiling_on_sc=False),
    )
    def deg_kernel(ei, ones, zeros, out, dstv, onesv, acc, sem):
        c = lax.axis_index("c")
        s = lax.axis_index("s")
        w, start, has_extra = _worker_slab(c, s)

        @pl.when(s == 0)
        def _():
            pltpu.sync_copy(zeros, acc)

        _load_idx_slab(ei, 1, start, has_extra, dstv)
        pltpu.sync_copy(ones, onesv)
        plsc.subcore_barrier()

        def fire(j, carry):
            pltpu.async_copy(onesv, acc.at[dstv.at[j]], sem, add=True)
            return carry

        lax.fori_loop(0, BASE, fire, 0)

        @pl.when(has_extra)
        def _():
            pltpu.async_copy(onesv, acc.at[dstv.at[BASE]], sem, add=True)

        def drain(j, carry):
            pltpu.make_async_copy(onesv, acc.at[dstv.at[j]], sem).wait()
            return carry

        lax.fori_loop(0, BASE, drain, 0)

        @pl.when(has_extra)
        def _():
            pltpu.make_async_copy(onesv, acc.at[dstv.at[BASE]], sem).wait()

        plsc.subcore_barrier()
        _writeback(acc, out, c, s)

    return deg_kernel


def _make_agg_kernel(d):
    @functools.partial(
        pl.kernel,
        out_type=jax.ShapeDtypeStruct((NC, N, d), jnp.float32),
        mesh=_sc_mesh(),
        scratch_types=[
            pltpu.VMEM((BASE + 1, CHUNK), jnp.int32),
            pltpu.VMEM((BASE + 1, CHUNK), jnp.int32),
            pltpu.VMEM((NBUF, CHUNK, d), jnp.float32),
            pltpu.VMEM_SHARED((N, d), jnp.float32),
            pltpu.SemaphoreType.DMA((NBUF,)),
            pltpu.SemaphoreType.DMA((NBUF,)),
        ],
        compiler_params=pltpu.CompilerParams(use_tc_tiling_on_sc=False),
    )
    def agg_kernel(y, ei, zeros, out, srcv, dstv, buf, acc, gsem, ssem):
        c = lax.axis_index("c")
        s = lax.axis_index("s")
        w, start, has_extra = _worker_slab(c, s)
        nch = BASE + has_extra.astype(jnp.int32)

        @pl.when(s == 0)
        def _():
            pltpu.sync_copy(zeros, acc)

        _load_idx_slab(ei, 0, start, has_extra, srcv)
        _load_idx_slab(ei, 1, start, has_extra, dstv)
        plsc.subcore_barrier()

        for b in range(NBUF):
            pltpu.async_copy(y.at[srcv.at[b]], buf.at[b], gsem.at[b])

        def body(g, carry):
            for b in range(NBUF):
                j = g * NBUF + b

                @pl.when(j < nch)
                def _(b=b, j=j):
                    pltpu.make_async_copy(
                        y.at[srcv.at[j]], buf.at[b], gsem.at[b]).wait()
                    pltpu.async_copy(buf.at[b], acc.at[dstv.at[j]],
                                     ssem.at[b], add=True)

                @pl.when(j + NBUF < nch)
                def _(b=b, j=j):
                    pltpu.make_async_copy(
                        buf.at[b], acc.at[dstv.at[j]], ssem.at[b]).wait()
                    pltpu.async_copy(
                        y.at[srcv.at[j + NBUF]], buf.at[b], gsem.at[b])

            return carry

        lax.fori_loop(0, (BASE + NBUF) // NBUF, body, 0)
        for b in range(NBUF):
            pltpu.make_async_copy(buf.at[b], acc.at[dstv.at[0]],
                                  ssem.at[b]).wait()
        plsc.subcore_barrier()
        _writeback(acc, out, c, s)

    return agg_kernel


_deg_kernel = _make_deg_kernel()
_agg64 = _make_agg_kernel(D_H)
_agg32 = _make_agg_kernel(D_OUT)

_R = 2000


def _dot(a, b):
    return jnp.dot(a, b, preferred_element_type=jnp.float32)


def _tc1_body(degp, x, w1, dis_o, y1_o):
    deg = degp[0, :, 0:1] + degp[1, :, 0:1] + 1.0
    dis = lax.rsqrt(deg)
    dis_o[...] = dis
    y1_o[...] = dis * _dot(x[...], w1[...])


def _tc1(degp, x, w1):
    return pl.pallas_call(
        _tc1_body,
        grid=(N // _R,),
        in_specs=[
            pl.BlockSpec((NC, _R, 8), lambda i: (0, i, 0)),
            pl.BlockSpec((_R, D_IN), lambda i: (i, 0)),
            pl.BlockSpec((D_IN, D_H), lambda i: (0, 0)),
        ],
        out_specs=[
            pl.BlockSpec((_R, 1), lambda i: (i, 0)),
            pl.BlockSpec((_R, D_H), lambda i: (i, 0)),
        ],
        out_shape=[
            jax.ShapeDtypeStruct((N, 1), jnp.float32),
            jax.ShapeDtypeStruct((N, D_H), jnp.float32),
        ],
    )(degp, x, w1)


def _tc2_body(agg, y1, dis, b1, w2, y2_o):
    h1 = jax.nn.relu(dis[...] * (agg[0] + agg[1] + y1[...]) + b1[...])
    y2_o[...] = dis[...] * _dot(h1, w2[...])


def _tc2(agg, y1, dis, b1, w2):
    return pl.pallas_call(
        _tc2_body,
        grid=(N // _R,),
        in_specs=[
            pl.BlockSpec((NC, _R, D_H), lambda i: (0, i, 0)),
            pl.BlockSpec((_R, D_H), lambda i: (i, 0)),
            pl.BlockSpec((_R, 1), lambda i: (i, 0)),
            pl.BlockSpec((1, D_H), lambda i: (0, 0)),
            pl.BlockSpec((D_H, D_OUT), lambda i: (0, 0)),
        ],
        out_specs=pl.BlockSpec((_R, D_OUT), lambda i: (i, 0)),
        out_shape=jax.ShapeDtypeStruct((N, D_OUT), jnp.float32),
    )(agg, y1, dis, b1, w2)


def _tc3_body(agg, y2, dis, b2, wr1, br1, wr2, br2, ws1, bs1, ws2, bs2,
              h_o, recon_o, score_o):
    h = dis[...] * (agg[0] + agg[1] + y2[...]) + b2[...]
    h_o[...] = h
    r = jax.nn.relu(_dot(h, wr1[...]) + br1[...])
    recon_o[...] = _dot(r, wr2[...]) + br2[...]
    sc = jax.nn.relu(_dot(h, ws1[...]) + bs1[...])
    score_o[...] = jax.nn.sigmoid(_dot(sc, ws2[...]) + bs2[...])


def _tc3(agg, y2, dis, b2, wr1, br1, wr2, br2, ws1, bs1, ws2, bs2):
    row = lambda i: (i, 0)
    full = lambda i: (0, 0)
    return pl.pallas_call(
        _tc3_body,
        grid=(N // _R,),
        in_specs=[
            pl.BlockSpec((NC, _R, D_OUT), lambda i: (0, i, 0)),
            pl.BlockSpec((_R, D_OUT), row),
            pl.BlockSpec((_R, 1), row),
            pl.BlockSpec((1, D_OUT), full),
            pl.BlockSpec((D_OUT, D_H), full),
            pl.BlockSpec((1, D_H), full),
            pl.BlockSpec((D_H, D_IN), full),
            pl.BlockSpec((1, D_IN), full),
            pl.BlockSpec((D_OUT, D_H // 2), full),
            pl.BlockSpec((1, D_H // 2), full),
            pl.BlockSpec((D_H // 2, 1), full),
            pl.BlockSpec((1, 1), full),
        ],
        out_specs=[
            pl.BlockSpec((_R, D_OUT), row),
            pl.BlockSpec((_R, D_IN), row),
            pl.BlockSpec((_R, 1), row),
        ],
        out_shape=[
            jax.ShapeDtypeStruct((N, D_OUT), jnp.float32),
            jax.ShapeDtypeStruct((N, D_IN), jnp.float32),
            jax.ShapeDtypeStruct((N, 1), jnp.float32),
        ],
    )(agg, y2, dis, b2, wr1, br1, wr2, br2, ws1, bs1, ws2, bs2)


def kernel(x, edge_index, W1, b1, W2, b2, Wr1, br1, Wr2, br2, Ws1, bs1, Ws2, bs2):
    ei = edge_index.reshape(2, NCHUNKS, CHUNK)

    degp = _deg_kernel(ei,
                       jnp.ones((CHUNK, 8), jnp.float32),
                       jnp.zeros((N, 8), jnp.float32))
    dis, y1 = _tc1(degp, x, W1)

    agg1 = _agg64(y1, ei, jnp.zeros((N, D_H), jnp.float32))
    y2 = _tc2(agg1, y1, dis, b1.reshape(1, D_H), W2)

    agg2 = _agg32(y2, ei, jnp.zeros((N, D_OUT), jnp.float32))
    h, recon, score = _tc3(agg2, y2, dis, b2.reshape(1, D_OUT),
                           Wr1, br1.reshape(1, D_H), Wr2, br2.reshape(1, D_IN),
                           Ws1, bs1.reshape(1, D_H // 2), Ws2, bs2.reshape(1, 1))
    return (h, recon, score)


# speedup vs baseline: 55.4265x; 1.1264x over previous
import functools

import jax
import jax.numpy as jnp
from jax import lax
from jax.experimental import pallas as pl
from jax.experimental.pallas import tpu as pltpu, tpu_sc as plsc

N = 10000
E = 320000
D_IN = 128
D_H = 64
D_OUT = 32

NC = 2
NS = 16
NW = NC * NS
CHUNK = 128
NCHUNKS = E // CHUNK
BASE = NCHUNKS // NW
EXTRA = NCHUNKS - BASE * NW
NBUF = 4
ROWS_PER_TILE = 624
REM_ROWS = N - NS * ROWS_PER_TILE


def _sc_mesh():
    return plsc.VectorSubcoreMesh(core_axis_name="c", subcore_axis_name="s")


def _worker_slab(c, s):
    w = s * NC + c
    start = BASE * w + jnp.minimum(w, EXTRA)
    has_extra = w < EXTRA
    return w, start, has_extra


def _load_idx_slab(ei, row, start, has_extra, idxv):
    pltpu.sync_copy(ei.at[row, pl.ds(start, BASE)], idxv.at[pl.ds(0, BASE)])

    @pl.when(has_extra)
    def _():
        pltpu.sync_copy(ei.at[row, pl.ds(start + BASE, 1)],
                        idxv.at[pl.ds(BASE, 1)])


def _writeback(acc, out, c, s):
    pltpu.sync_copy(
        acc.at[pl.ds(s * ROWS_PER_TILE, ROWS_PER_TILE)],
        out.at[c, pl.ds(s * ROWS_PER_TILE, ROWS_PER_TILE)],
    )

    @pl.when(s == NS - 1)
    def _():
        pltpu.sync_copy(
            acc.at[pl.ds(NS * ROWS_PER_TILE, REM_ROWS)],
            out.at[c, pl.ds(NS * ROWS_PER_TILE, REM_ROWS)],
        )


def _make_deg_kernel():
    @functools.partial(
        pl.kernel,
        out_type=jax.ShapeDtypeStruct((NC, N, 8), jnp.float32),
        mesh=_sc_mesh(),
        scratch_types=[
            pltpu.VMEM((BASE + 1, CHUNK), jnp.int32),
            pltpu.VMEM((CHUNK, 8), jnp.float32),
            pltpu.VMEM_SHARED((N, 8), jnp.float32),
            pltpu.SemaphoreType.DMA,
        ],
        compiler_params=pltpu.CompilerParams(use_tc_tiling_on_sc=False),
    )
    def deg_kernel(ei, ones, zeros, out, dstv, onesv, acc, sem):
        c = lax.axis_index("c")
        s = lax.axis_index("s")
        w, start, has_extra = _worker_slab(c, s)

        @pl.when(s == 0)
        def _():
            pltpu.sync_copy(zeros, acc)

        _load_idx_slab(ei, 1, start, has_extra, dstv)
        pltpu.sync_copy(ones, onesv)
        plsc.subcore_barrier()

        def fire(j, carry):
            pltpu.async_copy(onesv, acc.at[dstv.at[j]], sem, add=True)
            return carry

        lax.fori_loop(0, BASE, fire, 0)

        @pl.when(has_extra)
        def _():
            pltpu.async_copy(onesv, acc.at[dstv.at[BASE]], sem, add=True)

        def drain(j, carry):
            pltpu.make_async_copy(onesv, acc.at[dstv.at[j]], sem).wait()
            return carry

        lax.fori_loop(0, BASE, drain, 0)

        @pl.when(has_extra)
        def _():
            pltpu.make_async_copy(onesv, acc.at[dstv.at[BASE]], sem).wait()

        plsc.subcore_barrier()
        _writeback(acc, out, c, s)

    return deg_kernel


def _make_agg_kernel(d):
    @functools.partial(
        pl.kernel,
        out_type=jax.ShapeDtypeStruct((NC, N, d), jnp.float32),
        mesh=_sc_mesh(),
        scratch_types=[
            pltpu.VMEM((BASE + 1, CHUNK), jnp.int32),
            pltpu.VMEM((BASE + 1, CHUNK), jnp.int32),
            pltpu.VMEM((NBUF, CHUNK, d), jnp.float32),
            pltpu.VMEM_SHARED((N, d), jnp.float32),
            pltpu.SemaphoreType.DMA((NBUF,)),
            pltpu.SemaphoreType.DMA((NBUF,)),
        ],
        compiler_params=pltpu.CompilerParams(use_tc_tiling_on_sc=False),
    )
    def agg_kernel(y, ei, zeros, out, srcv, dstv, buf, acc, gsem, ssem):
        c = lax.axis_index("c")
        s = lax.axis_index("s")
        w, start, has_extra = _worker_slab(c, s)
        nch = BASE + has_extra.astype(jnp.int32)

        @pl.when(s == 0)
        def _():
            pltpu.sync_copy(zeros, acc)

        _load_idx_slab(ei, 0, start, has_extra, srcv)
        _load_idx_slab(ei, 1, start, has_extra, dstv)
        plsc.subcore_barrier()

        for b in range(NBUF):
            pltpu.async_copy(y.at[srcv.at[b]], buf.at[b], gsem.at[b])

        def body(g, carry):
            for b in range(NBUF):
                j = g * NBUF + b

                @pl.when(j < nch)
                def _(b=b, j=j):
                    pltpu.make_async_copy(
                        y.at[srcv.at[j]], buf.at[b], gsem.at[b]).wait()
                    pltpu.async_copy(buf.at[b], acc.at[dstv.at[j]],
                                     ssem.at[b], add=True)

                @pl.when(j + NBUF < nch)
                def _(b=b, j=j):
                    pltpu.make_async_copy(
                        buf.at[b], acc.at[dstv.at[j]], ssem.at[b]).wait()
                    pltpu.async_copy(
                        y.at[srcv.at[j + NBUF]], buf.at[b], gsem.at[b])

            return carry

        lax.fori_loop(0, (BASE + NBUF) // NBUF, body, 0)
        for b in range(NBUF):
            pltpu.make_async_copy(buf.at[b], acc.at[dstv.at[0]],
                                  ssem.at[b]).wait()
        plsc.subcore_barrier()
        _writeback(acc, out, c, s)

    return agg_kernel


_deg_kernel = _make_deg_kernel()
_agg64 = _make_agg_kernel(D_H)
_agg32 = _make_agg_kernel(D_OUT)

_R = 2048
_G = (N + _R - 1) // _R
_P = 4
_RP = _R // _P
_NP = N // _P


def _dot(a, b):
    return jnp.dot(a, b, preferred_element_type=jnp.float32)


def _blkdiag4(w):
    z = jnp.zeros_like(w)
    return jnp.concatenate([
        jnp.concatenate([w, z, z, z], axis=1),
        jnp.concatenate([z, w, z, z], axis=1),
        jnp.concatenate([z, z, w, z], axis=1),
        jnp.concatenate([z, z, z, w], axis=1),
    ], axis=0)


def _expander(d):
    return jnp.kron(jnp.eye(4, dtype=jnp.float32),
                    jnp.ones((1, d), jnp.float32))


def _selector():
    e0 = jnp.zeros((8, 1), jnp.float32).at[0, 0].set(1.0)
    return jnp.kron(jnp.eye(4, dtype=jnp.float32), e0.T).T


def _tc1_body(degp, x, w1b, s8, e64, dis_o, y1_o):
    deg4 = _dot(degp[0] + degp[1], s8[...]) + 1.0
    dis4 = lax.rsqrt(deg4)
    dis_o[...] = dis4
    y1_o[...] = _dot(dis4, e64[...]) * _dot(x[...], w1b[...])


def _tc1(degp, x, w1b, s8, e64):
    return pl.pallas_call(
        _tc1_body,
        grid=(_G,),
        in_specs=[
            pl.BlockSpec((NC, _RP, 32), lambda i: (0, i, 0)),
            pl.BlockSpec((_RP, 4 * D_IN), lambda i: (i, 0)),
            pl.BlockSpec((4 * D_IN, 4 * D_H), lambda i: (0, 0)),
            pl.BlockSpec((32, 4), lambda i: (0, 0)),
            pl.BlockSpec((4, 4 * D_H), lambda i: (0, 0)),
        ],
        out_specs=[
            pl.BlockSpec((_RP, 4), lambda i: (i, 0)),
            pl.BlockSpec((_RP, 4 * D_H), lambda i: (i, 0)),
        ],
        out_shape=[
            jax.ShapeDtypeStruct((_NP, 4), jnp.float32),
            jax.ShapeDtypeStruct((_NP, 4 * D_H), jnp.float32),
        ],
    )(degp, x, w1b, s8, e64)


def _tc2_body(agg, y1, dis, b1p, w2b, e64, e32, y2_o):
    de = _dot(dis[...], e64[...])
    h1 = jax.nn.relu(de * (agg[0] + agg[1] + y1[...]) + b1p[...])
    y2_o[...] = _dot(dis[...], e32[...]) * _dot(h1, w2b[...])


def _tc2(agg, y1, dis, b1p, w2b, e64, e32):
    return pl.pallas_call(
        _tc2_body,
        grid=(_G,),
        in_specs=[
            pl.BlockSpec((NC, _RP, 4 * D_H), lambda i: (0, i, 0)),
            pl.BlockSpec((_RP, 4 * D_H), lambda i: (i, 0)),
            pl.BlockSpec((_RP, 4), lambda i: (i, 0)),
            pl.BlockSpec((1, 4 * D_H), lambda i: (0, 0)),
            pl.BlockSpec((4 * D_H, 4 * D_OUT), lambda i: (0, 0)),
            pl.BlockSpec((4, 4 * D_H), lambda i: (0, 0)),
            pl.BlockSpec((4, 4 * D_OUT), lambda i: (0, 0)),
        ],
        out_specs=pl.BlockSpec((_RP, 4 * D_OUT), lambda i: (i, 0)),
        out_shape=jax.ShapeDtypeStruct((_NP, 4 * D_OUT), jnp.float32),
    )(agg, y1, dis, b1p, w2b, e64, e32)


def _tc3_body(agg, y2, dis, e32, b2p, wr1b, br1p, wr2b, br2p,
              ws1b, bs1p, ws2b, bs2p, h_o, recon_o, score_o):
    de = _dot(dis[...], e32[...])
    h = de * (agg[0] + agg[1] + y2[...]) + b2p[...]
    h_o[...] = h
    r = jax.nn.relu(_dot(h, wr1b[...]) + br1p[...])
    recon_o[...] = _dot(r, wr2b[...]) + br2p[...]
    sc = jax.nn.relu(_dot(h, ws1b[...]) + bs1p[...])
    score_o[...] = jax.nn.sigmoid(_dot(sc, ws2b[...]) + bs2p[...])


def _tc3(agg, y2, dis, e32, b2p, wr1b, br1p, wr2b, br2p, ws1b, bs1p, ws2b, bs2p):
    row = lambda i: (i, 0)
    full = lambda i: (0, 0)
    return pl.pallas_call(
        _tc3_body,
        grid=(_G,),
        in_specs=[
            pl.BlockSpec((NC, _RP, 4 * D_OUT), lambda i: (0, i, 0)),
            pl.BlockSpec((_RP, 4 * D_OUT), row),
            pl.BlockSpec((_RP, 4), row),
            pl.BlockSpec((4, 4 * D_OUT), full),
            pl.BlockSpec((1, 4 * D_OUT), full),
            pl.BlockSpec((4 * D_OUT, 4 * D_H), full),
            pl.BlockSpec((1, 4 * D_H), full),
            pl.BlockSpec((4 * D_H, 4 * D_IN), full),
            pl.BlockSpec((1, 4 * D_IN), full),
            pl.BlockSpec((4 * D_OUT, 4 * (D_H // 2)), full),
            pl.BlockSpec((1, 4 * (D_H // 2)), full),
            pl.BlockSpec((4 * (D_H // 2), 4), full),
            pl.BlockSpec((1, 4), full),
        ],
        out_specs=[
            pl.BlockSpec((_RP, 4 * D_OUT), row),
            pl.BlockSpec((_RP, 4 * D_IN), row),
            pl.BlockSpec((_RP, 4), row),
        ],
        out_shape=[
            jax.ShapeDtypeStruct((_NP, 4 * D_OUT), jnp.float32),
            jax.ShapeDtypeStruct((_NP, 4 * D_IN), jnp.float32),
            jax.ShapeDtypeStruct((_NP, 4), jnp.float32),
        ],
    )(agg, y2, dis, e32, b2p, wr1b, br1p, wr2b, br2p, ws1b, bs1p, ws2b, bs2p)


def _tile4(b):
    return jnp.tile(b, 4).reshape(1, 4 * b.shape[0])


def kernel(x, edge_index, W1, b1, W2, b2, Wr1, br1, Wr2, br2, Ws1, bs1, Ws2, bs2):
    ei = edge_index.reshape(2, NCHUNKS, CHUNK)
    s8 = _selector()
    e64 = _expander(D_H)
    e32 = _expander(D_OUT)

    degp = _deg_kernel(ei,
                       jnp.ones((CHUNK, 8), jnp.float32),
                       jnp.zeros((N, 8), jnp.float32))
    dis, y1 = _tc1(degp.reshape(NC, _NP, 32), x.reshape(_NP, 4 * D_IN),
                   _blkdiag4(W1), s8, e64)

    agg1 = _agg64(y1.reshape(N, D_H), ei, jnp.zeros((N, D_H), jnp.float32))
    y2 = _tc2(agg1.reshape(NC, _NP, 4 * D_H), y1, dis, _tile4(b1),
              _blkdiag4(W2), e64, e32)

    agg2 = _agg32(y2.reshape(N, D_OUT), ei, jnp.zeros((N, D_OUT), jnp.float32))
    h, recon, score = _tc3(agg2.reshape(NC, _NP, 4 * D_OUT), y2, dis, e32,
                           _tile4(b2), _blkdiag4(Wr1), _tile4(br1),
                           _blkdiag4(Wr2), _tile4(br2), _blkdiag4(Ws1),
                           _tile4(bs1), _blkdiag4(Ws2), _tile4(bs2))
    return (h.reshape(N, D_OUT), recon.reshape(N, D_IN), score.reshape(N, 1))


# speedup vs baseline: 56.3387x; 1.0165x over previous
import functools

import jax
import jax.numpy as jnp
from jax import lax
from jax.experimental import pallas as pl
from jax.experimental.pallas import tpu as pltpu, tpu_sc as plsc

N = 10000
E = 320000
D_IN = 128
D_H = 64
D_OUT = 32

NC = 2
NS = 16
NW = NC * NS
CHUNK = 128
NCHUNKS = E // CHUNK
BASE = NCHUNKS // NW
EXTRA = NCHUNKS - BASE * NW
NBUF = 6
ROWS_PER_TILE = 624
REM_ROWS = N - NS * ROWS_PER_TILE


def _sc_mesh():
    return plsc.VectorSubcoreMesh(core_axis_name="c", subcore_axis_name="s")


def _worker_slab(c, s):
    w = s * NC + c
    start = BASE * w + jnp.minimum(w, EXTRA)
    has_extra = w < EXTRA
    return w, start, has_extra


def _load_idx_slab(ei, row, start, has_extra, idxv):
    pltpu.sync_copy(ei.at[row, pl.ds(start, BASE)], idxv.at[pl.ds(0, BASE)])

    @pl.when(has_extra)
    def _():
        pltpu.sync_copy(ei.at[row, pl.ds(start + BASE, 1)],
                        idxv.at[pl.ds(BASE, 1)])


def _writeback(acc, out, c, s):
    pltpu.sync_copy(
        acc.at[pl.ds(s * ROWS_PER_TILE, ROWS_PER_TILE)],
        out.at[c, pl.ds(s * ROWS_PER_TILE, ROWS_PER_TILE)],
    )

    @pl.when(s == NS - 1)
    def _():
        pltpu.sync_copy(
            acc.at[pl.ds(NS * ROWS_PER_TILE, REM_ROWS)],
            out.at[c, pl.ds(NS * ROWS_PER_TILE, REM_ROWS)],
        )


def _make_deg_kernel():
    @functools.partial(
        pl.kernel,
        out_type=jax.ShapeDtypeStruct((NC, N, 8), jnp.float32),
        mesh=_sc_mesh(),
        scratch_types=[
            pltpu.VMEM((BASE + 1, CHUNK), jnp.int32),
            pltpu.VMEM((CHUNK, 8), jnp.float32),
            pltpu.VMEM_SHARED((N, 8), jnp.float32),
            pltpu.SemaphoreType.DMA,
        ],
        compiler_params=pltpu.CompilerParams(use_tc_tiling_on_sc=False),
    )
    def deg_kernel(ei, ones, zeros, out, dstv, onesv, acc, sem):
        c = lax.axis_index("c")
        s = lax.axis_index("s")
        w, start, has_extra = _worker_slab(c, s)

        @pl.when(s == 0)
        def _():
            pltpu.sync_copy(zeros, acc)

        _load_idx_slab(ei, 1, start, has_extra, dstv)
        pltpu.sync_copy(ones, onesv)
        plsc.subcore_barrier()

        def fire(j, carry):
            pltpu.async_copy(onesv, acc.at[dstv.at[j]], sem, add=True)
            return carry

        lax.fori_loop(0, BASE, fire, 0)

        @pl.when(has_extra)
        def _():
            pltpu.async_copy(onesv, acc.at[dstv.at[BASE]], sem, add=True)

        def drain(j, carry):
            pltpu.make_async_copy(onesv, acc.at[dstv.at[j]], sem).wait()
            return carry

        lax.fori_loop(0, BASE, drain, 0)

        @pl.when(has_extra)
        def _():
            pltpu.make_async_copy(onesv, acc.at[dstv.at[BASE]], sem).wait()

        plsc.subcore_barrier()
        _writeback(acc, out, c, s)

    return deg_kernel


def _make_agg_kernel(d):
    @functools.partial(
        pl.kernel,
        out_type=jax.ShapeDtypeStruct((NC, N, d), jnp.float32),
        mesh=_sc_mesh(),
        scratch_types=[
            pltpu.VMEM((BASE + 1, CHUNK), jnp.int32),
            pltpu.VMEM((BASE + 1, CHUNK), jnp.int32),
            pltpu.VMEM((NBUF, CHUNK, d), jnp.float32),
            pltpu.VMEM_SHARED((N, d), jnp.float32),
            pltpu.SemaphoreType.DMA((NBUF,)),
            pltpu.SemaphoreType.DMA((NBUF,)),
        ],
        compiler_params=pltpu.CompilerParams(use_tc_tiling_on_sc=False),
    )
    def agg_kernel(y, ei, zeros, out, srcv, dstv, buf, acc, gsem, ssem):
        c = lax.axis_index("c")
        s = lax.axis_index("s")
        w, start, has_extra = _worker_slab(c, s)
        nch = BASE + has_extra.astype(jnp.int32)

        @pl.when(s == 0)
        def _():
            pltpu.sync_copy(zeros, acc)

        _load_idx_slab(ei, 0, start, has_extra, srcv)
        _load_idx_slab(ei, 1, start, has_extra, dstv)
        plsc.subcore_barrier()

        for b in range(NBUF):
            pltpu.async_copy(y.at[srcv.at[b]], buf.at[b], gsem.at[b])

        def body(g, carry):
            for b in range(NBUF):
                j = g * NBUF + b

                @pl.when(j < nch)
                def _(b=b, j=j):
                    pltpu.make_async_copy(
                        y.at[srcv.at[j]], buf.at[b], gsem.at[b]).wait()
                    pltpu.async_copy(buf.at[b], acc.at[dstv.at[j]],
                                     ssem.at[b], add=True)

                @pl.when(j + NBUF < nch)
                def _(b=b, j=j):
                    pltpu.make_async_copy(
                        buf.at[b], acc.at[dstv.at[j]], ssem.at[b]).wait()
                    pltpu.async_copy(
                        y.at[srcv.at[j + NBUF]], buf.at[b], gsem.at[b])

            return carry

        lax.fori_loop(0, (BASE + NBUF) // NBUF, body, 0)
        for b in range(NBUF):
            pltpu.make_async_copy(buf.at[b], acc.at[dstv.at[0]],
                                  ssem.at[b]).wait()
        plsc.subcore_barrier()
        _writeback(acc, out, c, s)

    return agg_kernel


_deg_kernel = _make_deg_kernel()
_agg64 = _make_agg_kernel(D_H)
_agg32 = _make_agg_kernel(D_OUT)

_R = 2048
_G = (N + _R - 1) // _R
_P = 4
_RP = _R // _P
_NP = N // _P


def _dot(a, b):
    return jnp.dot(a, b, preferred_element_type=jnp.float32)


def _blkdiag4(w):
    z = jnp.zeros_like(w)
    return jnp.concatenate([
        jnp.concatenate([w, z, z, z], axis=1),
        jnp.concatenate([z, w, z, z], axis=1),
        jnp.concatenate([z, z, w, z], axis=1),
        jnp.concatenate([z, z, z, w], axis=1),
    ], axis=0)


def _expander(d):
    return jnp.kron(jnp.eye(4, dtype=jnp.float32),
                    jnp.ones((1, d), jnp.float32))


def _selector():
    e0 = jnp.zeros((8, 1), jnp.float32).at[0, 0].set(1.0)
    return jnp.kron(jnp.eye(4, dtype=jnp.float32), e0.T).T


def _tc1_body(degp, x, w1b, s8, e64, dis_o, y1_o):
    deg4 = _dot(degp[0] + degp[1], s8[...]) + 1.0
    dis4 = lax.rsqrt(deg4)
    dis_o[...] = dis4
    y1_o[...] = _dot(dis4, e64[...]) * _dot(x[...], w1b[...])


def _tc1(degp, x, w1b, s8, e64):
    return pl.pallas_call(
        _tc1_body,
        grid=(_G,),
        in_specs=[
            pl.BlockSpec((NC, _RP, 32), lambda i: (0, i, 0)),
            pl.BlockSpec((_RP, 4 * D_IN), lambda i: (i, 0)),
            pl.BlockSpec((4 * D_IN, 4 * D_H), lambda i: (0, 0)),
            pl.BlockSpec((32, 4), lambda i: (0, 0)),
            pl.BlockSpec((4, 4 * D_H), lambda i: (0, 0)),
        ],
        out_specs=[
            pl.BlockSpec((_RP, 4), lambda i: (i, 0)),
            pl.BlockSpec((_RP, 4 * D_H), lambda i: (i, 0)),
        ],
        out_shape=[
            jax.ShapeDtypeStruct((_NP, 4), jnp.float32),
            jax.ShapeDtypeStruct((_NP, 4 * D_H), jnp.float32),
        ],
    )(degp, x, w1b, s8, e64)


def _tc2_body(agg, y1, dis, b1p, w2b, e64, e32, y2_o):
    de = _dot(dis[...], e64[...])
    h1 = jax.nn.relu(de * (agg[0] + agg[1] + y1[...]) + b1p[...])
    y2_o[...] = _dot(dis[...], e32[...]) * _dot(h1, w2b[...])


def _tc2(agg, y1, dis, b1p, w2b, e64, e32):
    return pl.pallas_call(
        _tc2_body,
        grid=(_G,),
        in_specs=[
            pl.BlockSpec((NC, _RP, 4 * D_H), lambda i: (0, i, 0)),
            pl.BlockSpec((_RP, 4 * D_H), lambda i: (i, 0)),
            pl.BlockSpec((_RP, 4), lambda i: (i, 0)),
            pl.BlockSpec((1, 4 * D_H), lambda i: (0, 0)),
            pl.BlockSpec((4 * D_H, 4 * D_OUT), lambda i: (0, 0)),
            pl.BlockSpec((4, 4 * D_H), lambda i: (0, 0)),
            pl.BlockSpec((4, 4 * D_OUT), lambda i: (0, 0)),
        ],
        out_specs=pl.BlockSpec((_RP, 4 * D_OUT), lambda i: (i, 0)),
        out_shape=jax.ShapeDtypeStruct((_NP, 4 * D_OUT), jnp.float32),
    )(agg, y1, dis, b1p, w2b, e64, e32)


def _tc3_body(agg, y2, dis, e32, b2p, wr1b, br1p, wr2b, br2p,
              ws1b, bs1p, ws2b, bs2p, h_o, recon_o, score_o):
    de = _dot(dis[...], e32[...])
    h = de * (agg[0] + agg[1] + y2[...]) + b2p[...]
    h_o[...] = h
    r = jax.nn.relu(_dot(h, wr1b[...]) + br1p[...])
    recon_o[...] = _dot(r, wr2b[...]) + br2p[...]
    sc = jax.nn.relu(_dot(h, ws1b[...]) + bs1p[...])
    score_o[...] = jax.nn.sigmoid(_dot(sc, ws2b[...]) + bs2p[...])


def _tc3(agg, y2, dis, e32, b2p, wr1b, br1p, wr2b, br2p, ws1b, bs1p, ws2b, bs2p):
    row = lambda i: (i, 0)
    full = lambda i: (0, 0)
    return pl.pallas_call(
        _tc3_body,
        grid=(_G,),
        in_specs=[
            pl.BlockSpec((NC, _RP, 4 * D_OUT), lambda i: (0, i, 0)),
            pl.BlockSpec((_RP, 4 * D_OUT), row),
            pl.BlockSpec((_RP, 4), row),
            pl.BlockSpec((4, 4 * D_OUT), full),
            pl.BlockSpec((1, 4 * D_OUT), full),
            pl.BlockSpec((4 * D_OUT, 4 * D_H), full),
            pl.BlockSpec((1, 4 * D_H), full),
            pl.BlockSpec((4 * D_H, 4 * D_IN), full),
            pl.BlockSpec((1, 4 * D_IN), full),
            pl.BlockSpec((4 * D_OUT, 4 * (D_H // 2)), full),
            pl.BlockSpec((1, 4 * (D_H // 2)), full),
            pl.BlockSpec((4 * (D_H // 2), 4), full),
            pl.BlockSpec((1, 4), full),
        ],
        out_specs=[
            pl.BlockSpec((_RP, 4 * D_OUT), row),
            pl.BlockSpec((_RP, 4 * D_IN), row),
            pl.BlockSpec((_RP, 4), row),
        ],
        out_shape=[
            jax.ShapeDtypeStruct((_NP, 4 * D_OUT), jnp.float32),
            jax.ShapeDtypeStruct((_NP, 4 * D_IN), jnp.float32),
            jax.ShapeDtypeStruct((_NP, 4), jnp.float32),
        ],
    )(agg, y2, dis, e32, b2p, wr1b, br1p, wr2b, br2p, ws1b, bs1p, ws2b, bs2p)


def _tile4(b):
    return jnp.tile(b, 4).reshape(1, 4 * b.shape[0])


def kernel(x, edge_index, W1, b1, W2, b2, Wr1, br1, Wr2, br2, Ws1, bs1, Ws2, bs2):
    ei = edge_index.reshape(2, NCHUNKS, CHUNK)
    s8 = _selector()
    e64 = _expander(D_H)
    e32 = _expander(D_OUT)

    degp = _deg_kernel(ei,
                       jnp.ones((CHUNK, 8), jnp.float32),
                       jnp.zeros((N, 8), jnp.float32))
    dis, y1 = _tc1(degp.reshape(NC, _NP, 32), x.reshape(_NP, 4 * D_IN),
                   _blkdiag4(W1), s8, e64)

    agg1 = _agg64(y1.reshape(N, D_H), ei, jnp.zeros((N, D_H), jnp.float32))
    y2 = _tc2(agg1.reshape(NC, _NP, 4 * D_H), y1, dis, _tile4(b1),
              _blkdiag4(W2), e64, e32)

    agg2 = _agg32(y2.reshape(N, D_OUT), ei, jnp.zeros((N, D_OUT), jnp.float32))
    h, recon, score = _tc3(agg2.reshape(NC, _NP, 4 * D_OUT), y2, dis, e32,
                           _tile4(b2), _blkdiag4(Wr1), _tile4(br1),
                           _blkdiag4(Wr2), _tile4(br2), _blkdiag4(Ws1),
                           _tile4(bs1), _blkdiag4(Ws2), _tile4(bs2))
    return (h.reshape(N, D_OUT), recon.reshape(N, D_IN), score.reshape(N, 1))
